# baseline (speedup 1.0000x reference)
import jax
import jax.numpy as jnp
from jax.experimental import pallas as pl

_ROWS, _VOCAB = 128, 100000
_RB = 8
_C = 256
_NCHUNK = 391
_TINY = 1.1754943508222875e-38
_BIG = 2 ** 30


def _threefry_bits(idx):
    ks0 = jnp.uint32(0)
    ks1 = jnp.uint32(42)
    ks2 = jnp.uint32(466689008)
    x1 = jnp.zeros_like(idx)
    x2 = idx + ks1
    rots = ((13, 15, 26, 6), (17, 29, 16, 24))
    inject = ((ks1, ks2, 1), (ks2, ks0, 2), (ks0, ks1, 3),
              (ks1, ks2, 4), (ks2, ks0, 5))
    for g in range(5):
        for r in rots[g % 2]:
            x1 = x1 + x2
            x2 = (x2 << jnp.uint32(r)) | (x2 >> jnp.uint32(32 - r))
            x2 = x2 ^ x1
        a, b, i = inject[g]
        x1 = x1 + a
        x2 = x2 + b + jnp.uint32(i)
    return x1 ^ x2


def _gumbel(idx):
    bits = _threefry_bits(idx)
    tiny = jnp.float32(_TINY)
    fb = (bits >> jnp.uint32(9)) | jnp.uint32(0x3F800000)
    u = jax.lax.bitcast_convert_type(fb, jnp.float32) - jnp.float32(1.0)
    u = jnp.maximum(tiny, u)
    return -jnp.log(-jnp.log(u))


def _body(scores_ref, out_ref):
    pid = pl.program_id(0)
    row0 = pid * _RB
    r = jax.lax.broadcasted_iota(jnp.int32, (_RB, _C), 0)
    ci = jax.lax.broadcasted_iota(jnp.int32, (_RB, _C), 1)
    flat_base = (row0 + r) * _VOCAB + ci

    def chunk_off(j):
        return pl.multiple_of(j * _C, _C)

    def scan_step(j, st):
        m1, i1m, m2, i2m = st
        off = chunk_off(j)
        col = ci + off
        g = _gumbel((flat_base + off).astype(jnp.uint32))
        s = scores_ref[:, pl.ds(off, _C)] + g
        s = jnp.where(col < _VOCAB, s, -jnp.inf)
        take1 = s > m1
        take2 = s > m2
        n_m2 = jnp.where(take1, m1, jnp.where(take2, s, m2))
        n_i2 = jnp.where(take1, i1m, jnp.where(take2, col, i2m))
        n_m1 = jnp.maximum(s, m1)
        n_i1 = jnp.where(take1, col, i1m)
        return n_m1, n_i1, n_m2, n_i2

    neg = jnp.float32(-jnp.inf)
    init = (jnp.full((_RB, _C), neg), jnp.full((_RB, _C), _BIG, jnp.int32),
            jnp.full((_RB, _C), neg), jnp.full((_RB, _C), _BIG, jnp.int32))
    m1, i1m, m2, i2m = jax.lax.fori_loop(0, _NCHUNK, scan_step, init,
                                         unroll=32)

    v1 = jnp.max(m1, axis=1, keepdims=True)
    i1 = jnp.min(jnp.where(m1 == v1, i1m, _BIG), axis=1, keepdims=True)
    chosen = (m1 == v1) & (i1m == i1)
    cand_v = jnp.where(chosen, m2, m1)
    cand_i = jnp.where(chosen, i2m, i1m)
    v2 = jnp.max(cand_v, axis=1, keepdims=True)
    i2 = jnp.min(jnp.where(cand_v == v2, cand_i, _BIG), axis=1, keepdims=True)

    one = jnp.float32(1.0)
    zero = jnp.float32(0.0)

    def write_step(j, carry):
        off = chunk_off(j)
        col = ci + off
        out_ref[:, pl.ds(off, _C)] = jnp.where(
            (col == i1) | (col == i2), one, zero)
        return carry

    jax.lax.fori_loop(0, _NCHUNK, write_step, 0, unroll=16)


def _sampler(scores, interpret=False):
    return pl.pallas_call(
        _body,
        grid=(_ROWS // _RB,),
        in_specs=[pl.BlockSpec((_RB, _VOCAB), lambda i: (i, 0))],
        out_specs=pl.BlockSpec((_RB, _VOCAB), lambda i: (i, 0)),
        out_shape=jax.ShapeDtypeStruct((_ROWS, _VOCAB), jnp.float32),
        interpret=interpret,
    )(scores)


def kernel(scores):
    return _sampler(scores.astype(jnp.float32))


# speedup vs baseline: 1.0278x; 1.0278x over previous
import jax
import jax.numpy as jnp
from jax.experimental import pallas as pl

_ROWS, _VOCAB = 128, 100000
_RB = 16
_C = 256
_NCHUNK = 391
_TINY = 1.1754943508222875e-38
_BIG = 2 ** 30


def _threefry_bits(idx):
    ks0 = jnp.uint32(0)
    ks1 = jnp.uint32(42)
    ks2 = jnp.uint32(466689008)
    x1 = jnp.zeros_like(idx)
    x2 = idx + ks1
    rots = ((13, 15, 26, 6), (17, 29, 16, 24))
    inject = ((ks1, ks2, 1), (ks2, ks0, 2), (ks0, ks1, 3),
              (ks1, ks2, 4), (ks2, ks0, 5))
    for g in range(5):
        for r in rots[g % 2]:
            x1 = x1 + x2
            x2 = (x2 << jnp.uint32(r)) | (x2 >> jnp.uint32(32 - r))
            x2 = x2 ^ x1
        a, b, i = inject[g]
        x1 = x1 + a
        x2 = x2 + b + jnp.uint32(i)
    return x1 ^ x2


def _gumbel(idx):
    bits = _threefry_bits(idx)
    tiny = jnp.float32(_TINY)
    fb = (bits >> jnp.uint32(9)) | jnp.uint32(0x3F800000)
    u = jax.lax.bitcast_convert_type(fb, jnp.float32) - jnp.float32(1.0)
    u = jnp.maximum(tiny, u)
    return -jnp.log(-jnp.log(u))


def _body(scores_ref, out_ref):
    pid = pl.program_id(0)
    row0 = pid * _RB
    r = jax.lax.broadcasted_iota(jnp.int32, (_RB, _C), 0)
    ci = jax.lax.broadcasted_iota(jnp.int32, (_RB, _C), 1)
    flat_base = (row0 + r) * _VOCAB + ci

    def chunk_off(j):
        return pl.multiple_of(j * _C, _C)

    def scan_step(j, st):
        m1, i1m, m2, i2m = st
        off = chunk_off(j)
        col = ci + off
        g = _gumbel((flat_base + off).astype(jnp.uint32))
        s = scores_ref[:, pl.ds(off, _C)] + g
        s = jnp.where(col < _VOCAB, s, -jnp.inf)
        take1 = s > m1
        take2 = s > m2
        n_m2 = jnp.where(take1, m1, jnp.where(take2, s, m2))
        n_i2 = jnp.where(take1, i1m, jnp.where(take2, col, i2m))
        n_m1 = jnp.maximum(s, m1)
        n_i1 = jnp.where(take1, col, i1m)
        return n_m1, n_i1, n_m2, n_i2

    neg = jnp.float32(-jnp.inf)
    init = (jnp.full((_RB, _C), neg), jnp.full((_RB, _C), _BIG, jnp.int32),
            jnp.full((_RB, _C), neg), jnp.full((_RB, _C), _BIG, jnp.int32))
    m1, i1m, m2, i2m = jax.lax.fori_loop(0, _NCHUNK, scan_step, init,
                                         unroll=32)

    v1 = jnp.max(m1, axis=1, keepdims=True)
    i1 = jnp.min(jnp.where(m1 == v1, i1m, _BIG), axis=1, keepdims=True)
    chosen = (m1 == v1) & (i1m == i1)
    cand_v = jnp.where(chosen, m2, m1)
    cand_i = jnp.where(chosen, i2m, i1m)
    v2 = jnp.max(cand_v, axis=1, keepdims=True)
    i2 = jnp.min(jnp.where(cand_v == v2, cand_i, _BIG), axis=1, keepdims=True)

    one = jnp.float32(1.0)
    zero = jnp.float32(0.0)

    def write_step(j, carry):
        off = chunk_off(j)
        col = ci + off
        out_ref[:, pl.ds(off, _C)] = jnp.where(
            (col == i1) | (col == i2), one, zero)
        return carry

    jax.lax.fori_loop(0, _NCHUNK, write_step, 0, unroll=16)


def _sampler(scores, interpret=False):
    return pl.pallas_call(
        _body,
        grid=(_ROWS // _RB,),
        in_specs=[pl.BlockSpec((_RB, _VOCAB), lambda i: (i, 0))],
        out_specs=pl.BlockSpec((_RB, _VOCAB), lambda i: (i, 0)),
        out_shape=jax.ShapeDtypeStruct((_ROWS, _VOCAB), jnp.float32),
        interpret=interpret,
    )(scores)


def kernel(scores):
    return _sampler(scores.astype(jnp.float32))


# speedup vs baseline: 1.0291x; 1.0013x over previous
import jax
import jax.numpy as jnp
from jax.experimental import pallas as pl

_ROWS, _VOCAB = 128, 100000
_RB = 32
_C = 256
_NCHUNK = 391
_TINY = 1.1754943508222875e-38
_BIG = 2 ** 30


def _threefry_bits(idx):
    ks0 = jnp.uint32(0)
    ks1 = jnp.uint32(42)
    ks2 = jnp.uint32(466689008)
    x1 = jnp.zeros_like(idx)
    x2 = idx + ks1
    rots = ((13, 15, 26, 6), (17, 29, 16, 24))
    inject = ((ks1, ks2, 1), (ks2, ks0, 2), (ks0, ks1, 3),
              (ks1, ks2, 4), (ks2, ks0, 5))
    for g in range(5):
        for r in rots[g % 2]:
            x1 = x1 + x2
            x2 = (x2 << jnp.uint32(r)) | (x2 >> jnp.uint32(32 - r))
            x2 = x2 ^ x1
        a, b, i = inject[g]
        x1 = x1 + a
        x2 = x2 + b + jnp.uint32(i)
    return x1 ^ x2


def _gumbel(idx):
    bits = _threefry_bits(idx)
    tiny = jnp.float32(_TINY)
    fb = (bits >> jnp.uint32(9)) | jnp.uint32(0x3F800000)
    u = jax.lax.bitcast_convert_type(fb, jnp.float32) - jnp.float32(1.0)
    u = jnp.maximum(tiny, u)
    return -jnp.log(-jnp.log(u))


def _body(scores_ref, out_ref):
    pid = pl.program_id(0)
    row0 = pid * _RB
    r = jax.lax.broadcasted_iota(jnp.int32, (_RB, _C), 0)
    ci = jax.lax.broadcasted_iota(jnp.int32, (_RB, _C), 1)
    flat_base = (row0 + r) * _VOCAB + ci

    def chunk_off(j):
        return pl.multiple_of(j * _C, _C)

    def scan_step(j, st):
        m1, i1m, m2, i2m = st
        off = chunk_off(j)
        col = ci + off
        g = _gumbel((flat_base + off).astype(jnp.uint32))
        s = scores_ref[:, pl.ds(off, _C)] + g
        s = jnp.where(col < _VOCAB, s, -jnp.inf)
        take1 = s > m1
        take2 = s > m2
        n_m2 = jnp.where(take1, m1, jnp.where(take2, s, m2))
        n_i2 = jnp.where(take1, i1m, jnp.where(take2, col, i2m))
        n_m1 = jnp.maximum(s, m1)
        n_i1 = jnp.where(take1, col, i1m)
        return n_m1, n_i1, n_m2, n_i2

    neg = jnp.float32(-jnp.inf)
    init = (jnp.full((_RB, _C), neg), jnp.full((_RB, _C), _BIG, jnp.int32),
            jnp.full((_RB, _C), neg), jnp.full((_RB, _C), _BIG, jnp.int32))
    m1, i1m, m2, i2m = jax.lax.fori_loop(0, _NCHUNK, scan_step, init,
                                         unroll=32)

    v1 = jnp.max(m1, axis=1, keepdims=True)
    i1 = jnp.min(jnp.where(m1 == v1, i1m, _BIG), axis=1, keepdims=True)
    chosen = (m1 == v1) & (i1m == i1)
    cand_v = jnp.where(chosen, m2, m1)
    cand_i = jnp.where(chosen, i2m, i1m)
    v2 = jnp.max(cand_v, axis=1, keepdims=True)
    i2 = jnp.min(jnp.where(cand_v == v2, cand_i, _BIG), axis=1, keepdims=True)

    one = jnp.float32(1.0)
    zero = jnp.float32(0.0)

    def write_step(j, carry):
        off = chunk_off(j)
        col = ci + off
        out_ref[:, pl.ds(off, _C)] = jnp.where(
            (col == i1) | (col == i2), one, zero)
        return carry

    jax.lax.fori_loop(0, _NCHUNK, write_step, 0, unroll=16)


def _sampler(scores, interpret=False):
    return pl.pallas_call(
        _body,
        grid=(_ROWS // _RB,),
        in_specs=[pl.BlockSpec((_RB, _VOCAB), lambda i: (i, 0))],
        out_specs=pl.BlockSpec((_RB, _VOCAB), lambda i: (i, 0)),
        out_shape=jax.ShapeDtypeStruct((_ROWS, _VOCAB), jnp.float32),
        interpret=interpret,
    )(scores)


def kernel(scores):
    return _sampler(scores.astype(jnp.float32))
